# Initial kernel scaffold; baseline (speedup 1.0000x reference)
import jax
import jax.numpy as jnp
from jax.experimental import pallas as pl


def kernel(x, edge_index, x_label, W_label, W1, b1, W2, b2):
    raise NotImplementedError("write your pallas kernel here")


# speedup vs baseline: 2.0434x; 2.0434x over previous
import functools

import jax
import jax.numpy as jnp
from jax import lax
from jax.experimental import pallas as pl
from jax.experimental.pallas import tpu as pltpu
from jax.experimental.pallas import tpu_sc as plsc

N_NODES = 10000
N_EDGES = 320000
F = 128

NC = 2
NS = 16
L = 16
NW = NC * NS
EPW = N_EDGES // NW
CH_B = 80
CH_D = 2560
FG = 16
RPG = F // FG

_DN_CONTRACT_MINOR = (((1,), (1,)), ((), ()))


def _node_tables_body(x_ref, wl_ref, w1_ref, cd_ref, cs_ref):
    x = x_ref[...]
    wl = wl_ref[...]
    a = wl[:, :F]
    b = wl[:, F:]
    w1 = w1_ref[...]
    cd0 = lax.dot_general(x, a - b, _DN_CONTRACT_MINOR,
                          preferred_element_type=jnp.float32)
    cs0 = lax.dot_general(x, b, _DN_CONTRACT_MINOR,
                          preferred_element_type=jnp.float32)
    cd_ref[...] = lax.dot_general(cd0, w1, _DN_CONTRACT_MINOR,
                                  preferred_element_type=jnp.float32)
    cs_ref[...] = lax.dot_general(cs0, w1, _DN_CONTRACT_MINOR,
                                  preferred_element_type=jnp.float32)


def _node_tables(x, w_label, w1):
    nb = 2000
    grid = (N_NODES // nb,)
    return pl.pallas_call(
        _node_tables_body,
        grid=grid,
        in_specs=[
            pl.BlockSpec((nb, F), lambda i: (i, 0)),
            pl.BlockSpec((F, 2 * F), lambda i: (0, 0)),
            pl.BlockSpec((F, F), lambda i: (0, 0)),
        ],
        out_specs=[
            pl.BlockSpec((nb, F), lambda i: (i, 0)),
            pl.BlockSpec((nb, F), lambda i: (i, 0)),
        ],
        out_shape=[
            jax.ShapeDtypeStruct((N_NODES, F), jnp.float32),
            jax.ShapeDtypeStruct((N_NODES, F), jnp.float32),
        ],
    )(x, w_label, w1)


def _gather_add_body(cd_hbm, cs_hbm, src_hbm, dst_hbm, g_hbm,
                     didx_v, sidx_v, cdr_v, csr_v, sem):
    wid = lax.axis_index("s") * NC + lax.axis_index("c")
    base_w = wid * EPW

    def chunk(ci, carry):
        base = base_w + ci * CH_B
        pltpu.sync_copy(dst_hbm.at[pl.ds(base, CH_B)], didx_v)
        pltpu.sync_copy(src_hbm.at[pl.ds(base, CH_B)], sidx_v)
        pltpu.async_copy(cd_hbm.at[didx_v], cdr_v, sem).wait()
        pltpu.async_copy(cs_hbm.at[sidx_v], csr_v, sem).wait()

        def row(e, c2):
            for j in range(F // L):
                s = pl.ds(j * L, L)
                cdr_v[e, s] = cdr_v[e, s] + csr_v[e, s]
            return c2

        lax.fori_loop(0, CH_B, row, 0)
        pltpu.sync_copy(cdr_v, g_hbm.at[pl.ds(base, CH_B), :])
        return carry

    lax.fori_loop(0, EPW // CH_B, chunk, 0)


def _gather_add(cd, cs, src, dst):
    mesh = plsc.VectorSubcoreMesh(
        core_axis_name="c", subcore_axis_name="s",
        num_cores=NC, num_subcores=NS)
    fn = pl.kernel(
        _gather_add_body,
        out_type=jax.ShapeDtypeStruct((N_EDGES, F), jnp.float32),
        mesh=mesh,
        compiler_params=pltpu.CompilerParams(needs_layout_passes=False),
        scratch_types=[
            pltpu.VMEM((CH_B,), jnp.int32),
            pltpu.VMEM((CH_B,), jnp.int32),
            pltpu.VMEM((CH_B, F), jnp.float32),
            pltpu.VMEM((CH_B, F), jnp.float32),
            pltpu.SemaphoreType.DMA,
        ],
    )
    return fn(cd, cs, src, dst)


def _edge_mlp_body(g_ref, xl_ref, w1_ref, b1_ref, w2_ref, b2_ref, mt_ref):
    t = lax.dot_general(xl_ref[...], w1_ref[...], _DN_CONTRACT_MINOR,
                        preferred_element_type=jnp.float32)
    h = jnp.maximum(g_ref[...] + t + b1_ref[...], 0.0)
    mt = lax.dot_general(w2_ref[...], h, _DN_CONTRACT_MINOR,
                         preferred_element_type=jnp.float32)
    mt_ref[...] = (mt + b2_ref[...]).reshape(FG, RPG, mt.shape[-1])


def _edge_mlp(g, x_label, w1, b1, w2, b2):
    eb = 2560
    grid = (N_EDGES // eb,)
    return pl.pallas_call(
        _edge_mlp_body,
        grid=grid,
        in_specs=[
            pl.BlockSpec((eb, F), lambda i: (i, 0)),
            pl.BlockSpec((eb, F), lambda i: (i, 0)),
            pl.BlockSpec((F, F), lambda i: (0, 0)),
            pl.BlockSpec((1, F), lambda i: (0, 0)),
            pl.BlockSpec((F, F), lambda i: (0, 0)),
            pl.BlockSpec((F, 1), lambda i: (0, 0)),
        ],
        out_specs=pl.BlockSpec((FG, RPG, eb), lambda i: (0, 0, i)),
        out_shape=jax.ShapeDtypeStruct((FG, RPG, N_EDGES), jnp.float32),
    )(g, x_label, w1, b1, w2, b2)


def _segmax_body(mt_hbm, dst_hbm, out_hbm, didx_v, mrow_v, acc_v, probe_v, sem):
    wid = lax.axis_index("s") * NC + lax.axis_index("c")
    fg = wid % FG
    half = wid // FG
    neg_inf = jnp.float32(float("-inf"))
    iota = lax.iota(jnp.int32, L)
    rconst = [jnp.full((L,), r, jnp.int32) for r in range(RPG)]
    n_chunks = N_EDGES // CH_D

    def init(i, c):
        for r in range(RPG):
            acc_v[r, pl.ds(i * L, L)] = jnp.full((L,), neg_inf, jnp.float32)
        return c

    lax.fori_loop(0, N_NODES // L, init, 0)

    def chunk(ci, carry):
        base = (2 * ci + half) * CH_D
        pltpu.sync_copy(dst_hbm.at[pl.ds(base, CH_D)], didx_v)
        pltpu.sync_copy(mt_hbm.at[fg, :, pl.ds(base, CH_D)], mrow_v)

        def vec(i, c2):
            s = pl.ds(i * L, L)
            dstv = didx_v[s]
            vals = [mrow_v[r, s] for r in range(RPG)]

            plsc.store_scatter(probe_v, [dstv], iota)
            got = plsc.load_gather(probe_v, [dstv])
            win = got == iota
            for r in range(RPG):
                cur = plsc.load_gather(acc_v, [rconst[r], dstv])
                plsc.store_scatter(acc_v, [rconst[r], dstv],
                                   jnp.maximum(cur, vals[r]), mask=win)
            pend = jnp.where(win, 0, 1)

            def cond(p):
                return jnp.max(p) > 0

            def body(p):
                m = p > 0
                plsc.store_scatter(probe_v, [dstv], iota, mask=m)
                got2 = plsc.load_gather(probe_v, [dstv])
                win2 = m & (got2 == iota)
                for r in range(RPG):
                    cur2 = plsc.load_gather(acc_v, [rconst[r], dstv])
                    plsc.store_scatter(acc_v, [rconst[r], dstv],
                                       jnp.maximum(cur2, vals[r]), mask=win2)
                return jnp.where(win2, 0, p)

            lax.while_loop(cond, body, pend)
            return c2

        lax.fori_loop(0, CH_D // L, vec, 0)
        return carry

    lax.fori_loop(0, (n_chunks + 1 - half) // 2, chunk, 0)
    pltpu.sync_copy(acc_v, out_hbm.at[half, pl.ds(fg * RPG, RPG), :])


def _segment_max(mt, dst):
    mesh = plsc.VectorSubcoreMesh(
        core_axis_name="c", subcore_axis_name="s",
        num_cores=NC, num_subcores=NS)
    fn = pl.kernel(
        _segmax_body,
        out_type=jax.ShapeDtypeStruct((2, F, N_NODES), jnp.float32),
        mesh=mesh,
        compiler_params=pltpu.CompilerParams(needs_layout_passes=False),
        scratch_types=[
            pltpu.VMEM((CH_D,), jnp.int32),
            pltpu.VMEM((RPG, CH_D), jnp.float32),
            pltpu.VMEM((RPG, N_NODES), jnp.float32),
            pltpu.VMEM((N_NODES,), jnp.int32),
            pltpu.SemaphoreType.DMA,
        ],
    )
    return fn(mt, dst)


def _merge_body(p_ref, out_ref):
    neg_inf = jnp.float32(float("-inf"))
    mx = jnp.maximum(p_ref[0], p_ref[1])
    out_ref[...] = jnp.where(mx == neg_inf, jnp.float32(0.0), mx)


def _merge_halves(p):
    return pl.pallas_call(
        _merge_body,
        grid=(1,),
        in_specs=[pl.BlockSpec((2, F, N_NODES), lambda i: (0, 0, 0))],
        out_specs=pl.BlockSpec((F, N_NODES), lambda i: (0, 0)),
        out_shape=jax.ShapeDtypeStruct((F, N_NODES), jnp.float32),
    )(p)


def kernel(x, edge_index, x_label, W_label, W1, b1, W2, b2):
    src = edge_index[0]
    dst = edge_index[1]
    cd, cs = _node_tables(x, W_label, W1)
    g = _gather_add(cd, cs, src, dst)
    mt = _edge_mlp(g, x_label, W1, b1.reshape(1, F), W2, b2.reshape(F, 1))
    p = _segment_max(mt, dst)
    outt = _merge_halves(p)
    return outt.T


# speedup vs baseline: 3.0606x; 1.4978x over previous
import functools

import jax
import jax.numpy as jnp
from jax import lax
from jax.experimental import pallas as pl
from jax.experimental.pallas import tpu as pltpu
from jax.experimental.pallas import tpu_sc as plsc

N_NODES = 10000
N_EDGES = 320000
F = 128

NC = 2
NS = 16
L = 16
NW = NC * NS
EPW = N_EDGES // NW
CH_B = 200
CH_D = 1280
FG = 16
RPG = F // FG

_DN_CONTRACT_MINOR = (((1,), (1,)), ((), ()))


def _node_tables_body(x_ref, wl_ref, w1_ref, cd_ref, cs_ref):
    x = x_ref[...]
    wl = wl_ref[...]
    a = wl[:, :F]
    b = wl[:, F:]
    w1 = w1_ref[...]
    cd0 = lax.dot_general(x, a - b, _DN_CONTRACT_MINOR,
                          preferred_element_type=jnp.float32)
    cs0 = lax.dot_general(x, b, _DN_CONTRACT_MINOR,
                          preferred_element_type=jnp.float32)
    cd_ref[...] = lax.dot_general(cd0, w1, _DN_CONTRACT_MINOR,
                                  preferred_element_type=jnp.float32)
    cs_ref[...] = lax.dot_general(cs0, w1, _DN_CONTRACT_MINOR,
                                  preferred_element_type=jnp.float32)


def _node_tables(x, w_label, w1):
    nb = 2000
    grid = (N_NODES // nb,)
    return pl.pallas_call(
        _node_tables_body,
        grid=grid,
        in_specs=[
            pl.BlockSpec((nb, F), lambda i: (i, 0)),
            pl.BlockSpec((F, 2 * F), lambda i: (0, 0)),
            pl.BlockSpec((F, F), lambda i: (0, 0)),
        ],
        out_specs=[
            pl.BlockSpec((nb, F), lambda i: (i, 0)),
            pl.BlockSpec((nb, F), lambda i: (i, 0)),
        ],
        out_shape=[
            jax.ShapeDtypeStruct((N_NODES, F), jnp.float32),
            jax.ShapeDtypeStruct((N_NODES, F), jnp.float32),
        ],
    )(x, w_label, w1)


NCH_B = EPW // CH_B


def _gather_add_body(cd_hbm, cs_hbm, src_hbm, dst_hbm, g_hbm,
                     didx_v, sidx_v, cdr_v, csr_v,
                     semi, semg0, semg1, semw0, semw1):
    semg = (semg0, semg1)
    semw = (semw0, semw1)
    wid = lax.axis_index("s") * NC + lax.axis_index("c")
    base_w = wid * EPW

    cpi0 = pltpu.async_copy(dst_hbm.at[pl.ds(base_w, EPW)], didx_v, semi)
    cpi1 = pltpu.async_copy(src_hbm.at[pl.ds(base_w, EPW)], sidx_v, semi)
    cpi0.wait()
    cpi1.wait()

    def start_gather(c, b):
        sl = pl.ds(c * CH_B, CH_B)
        pltpu.async_copy(cd_hbm.at[didx_v.at[sl]], cdr_v.at[b], semg[b])
        pltpu.async_copy(cs_hbm.at[sidx_v.at[sl]], csr_v.at[b], semg[b])

    def wait_gather(c, b):
        sl = pl.ds(c * CH_B, CH_B)
        pltpu.make_async_copy(cd_hbm.at[didx_v.at[sl]], cdr_v.at[b],
                              semg[b]).wait()
        pltpu.make_async_copy(cs_hbm.at[sidx_v.at[sl]], csr_v.at[b],
                              semg[b]).wait()

    def wait_write(c, b):
        pltpu.make_async_copy(cdr_v.at[b],
                              g_hbm.at[pl.ds(base_w + c * CH_B, CH_B), :],
                              semw[b]).wait()

    start_gather(0, 0)

    def process(c, b, first, last):
        b2 = 1 - b
        wait_gather(c, b)
        if not first:
            wait_write(c - 1, b2)
        if not last:
            start_gather(c + 1, b2)

        def row(e, c2):
            for j in range(F // L):
                s = pl.ds(j * L, L)
                cdr_v[b, e, s] = cdr_v[b, e, s] + csr_v[b, e, s]
            return c2

        lax.fori_loop(0, CH_B, row, 0)
        pltpu.async_copy(cdr_v.at[b],
                         g_hbm.at[pl.ds(base_w + c * CH_B, CH_B), :], semw[b])

    process(0, 0, True, False)

    def pair(ci, carry):
        c = 1 + 2 * ci
        process(c, 1, False, False)
        process(c + 1, 0, False, False)
        return carry

    lax.fori_loop(0, (NCH_B - 2) // 2, pair, 0)
    process(NCH_B - 1, 1, False, True)
    wait_write(NCH_B - 1, 1)


def _gather_add(cd, cs, src, dst):
    mesh = plsc.VectorSubcoreMesh(
        core_axis_name="c", subcore_axis_name="s",
        num_cores=NC, num_subcores=NS)
    fn = pl.kernel(
        _gather_add_body,
        out_type=jax.ShapeDtypeStruct((N_EDGES, F), jnp.float32),
        mesh=mesh,
        compiler_params=pltpu.CompilerParams(needs_layout_passes=False),
        scratch_types=[
            pltpu.VMEM((EPW,), jnp.int32),
            pltpu.VMEM((EPW,), jnp.int32),
            pltpu.VMEM((2, CH_B, F), jnp.float32),
            pltpu.VMEM((2, CH_B, F), jnp.float32),
            pltpu.SemaphoreType.DMA,
            pltpu.SemaphoreType.DMA,
            pltpu.SemaphoreType.DMA,
            pltpu.SemaphoreType.DMA,
            pltpu.SemaphoreType.DMA,
        ],
    )
    return fn(cd, cs, src, dst)


def _edge_mlp_body(g_ref, xl_ref, w1_ref, b1_ref, w2_ref, b2_ref, mt_ref):
    t = lax.dot_general(xl_ref[...], w1_ref[...], _DN_CONTRACT_MINOR,
                        preferred_element_type=jnp.float32)
    h = jnp.maximum(g_ref[...] + t + b1_ref[...], 0.0)
    mt = lax.dot_general(w2_ref[...], h, _DN_CONTRACT_MINOR,
                         preferred_element_type=jnp.float32)
    mt_ref[...] = (mt + b2_ref[...]).reshape(FG, RPG, mt.shape[-1])


def _edge_mlp(g, x_label, w1, b1, w2, b2):
    eb = 2560
    grid = (N_EDGES // eb,)
    return pl.pallas_call(
        _edge_mlp_body,
        grid=grid,
        in_specs=[
            pl.BlockSpec((eb, F), lambda i: (i, 0)),
            pl.BlockSpec((eb, F), lambda i: (i, 0)),
            pl.BlockSpec((F, F), lambda i: (0, 0)),
            pl.BlockSpec((1, F), lambda i: (0, 0)),
            pl.BlockSpec((F, F), lambda i: (0, 0)),
            pl.BlockSpec((F, 1), lambda i: (0, 0)),
        ],
        out_specs=pl.BlockSpec((FG, RPG, eb), lambda i: (0, 0, i)),
        out_shape=jax.ShapeDtypeStruct((FG, RPG, N_EDGES), jnp.float32),
    )(g, x_label, w1, b1, w2, b2)


N_CH_D = N_EDGES // CH_D


def _segmax_body(mt_hbm, dst_hbm, out_hbm, didx_v, mrow_v, acc_v, probe_v,
                 semd0, semd1):
    semd = (semd0, semd1)
    wid = lax.axis_index("s") * NC + lax.axis_index("c")
    fg = wid % FG
    half = wid // FG
    neg_inf = jnp.float32(float("-inf"))
    iota = lax.iota(jnp.int32, L)
    rconst = [jnp.full((L,), r, jnp.int32) for r in range(RPG)]

    def init(i, c):
        for r in range(RPG):
            acc_v[r, pl.ds(i * L, L)] = jnp.full((L,), neg_inf, jnp.float32)
        return c

    lax.fori_loop(0, N_NODES // L, init, 0)

    def start_dma(k, b):
        base = (2 * k + half) * CH_D
        pltpu.async_copy(dst_hbm.at[pl.ds(base, CH_D)], didx_v.at[b], semd[b])
        pltpu.async_copy(mt_hbm.at[fg, :, pl.ds(base, CH_D)], mrow_v.at[b],
                         semd[b])

    def wait_dma(k, b):
        base = (2 * k + half) * CH_D
        pltpu.make_async_copy(dst_hbm.at[pl.ds(base, CH_D)], didx_v.at[b],
                              semd[b]).wait()
        pltpu.make_async_copy(mt_hbm.at[fg, :, pl.ds(base, CH_D)],
                              mrow_v.at[b], semd[b]).wait()

    def process(k, b, last):
        b2 = 1 - b
        wait_dma(k, b)
        if not last:
            start_dma(k + 1, b2)

        def vec(i, c2):
            s = pl.ds(i * L, L)
            dstv = didx_v[b, s]
            vals = [mrow_v[b, r, s] for r in range(RPG)]

            plsc.store_scatter(probe_v, [dstv], iota)
            got = plsc.load_gather(probe_v, [dstv])
            win = got == iota
            for r in range(RPG):
                cur = plsc.load_gather(acc_v, [rconst[r], dstv])
                plsc.store_scatter(acc_v, [rconst[r], dstv],
                                   jnp.maximum(cur, vals[r]), mask=win)
            pend = jnp.where(win, 0, 1)

            def cond(p):
                return jnp.max(p) > 0

            def body(p):
                m = p > 0
                plsc.store_scatter(probe_v, [dstv], iota, mask=m)
                got2 = plsc.load_gather(probe_v, [dstv])
                win2 = m & (got2 == iota)
                for r in range(RPG):
                    cur2 = plsc.load_gather(acc_v, [rconst[r], dstv])
                    plsc.store_scatter(acc_v, [rconst[r], dstv],
                                       jnp.maximum(cur2, vals[r]), mask=win2)
                return jnp.where(win2, 0, p)

            lax.while_loop(cond, body, pend)
            return c2

        lax.fori_loop(0, CH_D // L, vec, 0)

    start_dma(0, 0)
    process(0, 0, False)

    def pair(ci, carry):
        k = 1 + 2 * ci
        process(k, 1, False)
        process(k + 1, 0, False)
        return carry

    lax.fori_loop(0, 61, pair, 0)
    process(123, 1, False)
    process(124, 0, True)
    pltpu.sync_copy(acc_v, out_hbm.at[half, pl.ds(fg * RPG, RPG), :])


def _segment_max(mt, dst):
    mesh = plsc.VectorSubcoreMesh(
        core_axis_name="c", subcore_axis_name="s",
        num_cores=NC, num_subcores=NS)
    fn = pl.kernel(
        _segmax_body,
        out_type=jax.ShapeDtypeStruct((2, F, N_NODES), jnp.float32),
        mesh=mesh,
        compiler_params=pltpu.CompilerParams(needs_layout_passes=False),
        scratch_types=[
            pltpu.VMEM((2, CH_D), jnp.int32),
            pltpu.VMEM((2, RPG, CH_D), jnp.float32),
            pltpu.VMEM((RPG, N_NODES), jnp.float32),
            pltpu.VMEM((N_NODES,), jnp.int32),
            pltpu.SemaphoreType.DMA,
            pltpu.SemaphoreType.DMA,
        ],
    )
    return fn(mt, dst)


def _merge_body(p_ref, out_ref):
    neg_inf = jnp.float32(float("-inf"))
    mx = jnp.maximum(p_ref[0], p_ref[1])
    out_ref[...] = jnp.where(mx == neg_inf, jnp.float32(0.0), mx)


def _merge_halves(p):
    return pl.pallas_call(
        _merge_body,
        grid=(1,),
        in_specs=[pl.BlockSpec((2, F, N_NODES), lambda i: (0, 0, 0))],
        out_specs=pl.BlockSpec((F, N_NODES), lambda i: (0, 0)),
        out_shape=jax.ShapeDtypeStruct((F, N_NODES), jnp.float32),
    )(p)


def kernel(x, edge_index, x_label, W_label, W1, b1, W2, b2):
    src = edge_index[0]
    dst = edge_index[1]
    cd, cs = _node_tables(x, W_label, W1)
    g = _gather_add(cd, cs, src, dst)
    mt = _edge_mlp(g, x_label, W1, b1.reshape(1, F), W2, b2.reshape(F, 1))
    p = _segment_max(mt, dst)
    outt = _merge_halves(p)
    return outt.T
